# baseline (speedup 1.0000x reference)
import dataclasses
import functools

import jax
import jax.numpy as jnp
from jax import lax
from jax.experimental import pallas as pl
from jax.experimental.pallas import tpu as pltpu
from jax.experimental.pallas import tpu_sc as plsc

_G = 64
_B = 2560

_INTERPRET = False


def _edge_batch_sc(node_batch_i32, src_i32):
    N = node_batch_i32.shape[0]
    E = src_i32.shape[0]
    NC, NS, L = 2, 16, 16
    NW = NC * NS
    bpw = E // NW
    mesh = plsc.VectorSubcoreMesh(core_axis_name="c", subcore_axis_name="s")
    cp = pltpu.CompilerParams()
    if "needs_layout_passes" in pltpu.CompilerParams.__dataclass_fields__:
        cp = dataclasses.replace(cp, needs_layout_passes=False)

    @functools.partial(
        pl.kernel, mesh=mesh,
        compiler_params=cp,
        out_type=jax.ShapeDtypeStruct((E,), jnp.int32),
        scratch_types=[
            pltpu.VMEM((N,), jnp.int32),
            pltpu.VMEM((bpw,), jnp.int32),
            pltpu.VMEM((bpw,), jnp.int32),
            pltpu.SemaphoreType.DMA,
        ],
    )
    def k(nb_hbm, src_hbm, eb_hbm, nb_v, src_v, eb_v, sem):
        wid = lax.axis_index("s") * NC + lax.axis_index("c")
        base = wid * bpw
        pltpu.async_copy(nb_hbm, nb_v, sem).wait()
        pltpu.async_copy(src_hbm.at[pl.ds(base, bpw)], src_v, sem).wait()

        @pl.loop(0, bpw, step=L)
        def _(i):
            idx = src_v[pl.ds(i, L)]
            eb_v[pl.ds(i, L)] = plsc.load_gather(nb_v, [idx])

        pltpu.async_copy(eb_v, eb_hbm.at[pl.ds(base, bpw)], sem).wait()

    return k(node_batch_i32, src_i32)


def kernel(edge_attr, edge_index, node_batch, Gamma_W, Gamma_b, Lambda_W):
    E, D = edge_attr.shape
    G, B = _G, _B
    NB = E // B
    assert E % B == 0

    src = edge_index[0].astype(jnp.int32)
    nb32 = node_batch.astype(jnp.int32)
    eb = _edge_batch_sc(nb32, src)
    eb_r = eb.reshape(NB, 1, B)
    starts = jnp.searchsorted(
        nb32, jnp.arange(G + 1, dtype=jnp.int32), side="left"
    ).astype(jnp.int32)
    smat = jnp.broadcast_to(starts[:G, None], (G, B))
    emat = jnp.broadcast_to(starts[1:, None], (G, B))
    src_r = src.reshape(NB, 1, B)
    lamT = Lambda_W.T
    gamT_bf = Gamma_W.T.astype(jnp.bfloat16)
    gb = Gamma_b.reshape(1, D)

    def _pool_body(src_ref, ea_ref, smat_ref, emat_ref, lamT_ref, gb_ref,
                   padj_ref, acc_ref):
        i = pl.program_id(0)

        @pl.when(i == 0)
        def _():
            acc_ref[...] = jnp.zeros_like(acc_ref)

        srcb = jnp.broadcast_to(src_ref[0], (G, B))
        ind = (srcb >= smat_ref[...]) & (srcb < emat_ref[...])
        indT = ind.astype(jnp.bfloat16)
        ea = ea_ref[...].astype(jnp.bfloat16)
        acc_ref[...] += jax.lax.dot_general(
            indT, ea, (((1,), (0,)), ((), ())),
            preferred_element_type=jnp.float32)

        @pl.when(i == NB - 1)
        def _():
            padj_ref[...] = jax.lax.dot_general(
                acc_ref[...], lamT_ref[...], (((1,), (0,)), ((), ())),
                preferred_element_type=jnp.float32) - gb_ref[...]

    padj = pl.pallas_call(
        _pool_body,
        grid=(NB,),
        in_specs=[
            pl.BlockSpec((1, 1, B), lambda i: (i, 0, 0)),
            pl.BlockSpec((B, D), lambda i: (i, 0)),
            pl.BlockSpec((G, B), lambda i: (0, 0)),
            pl.BlockSpec((G, B), lambda i: (0, 0)),
            pl.BlockSpec((D, D), lambda i: (0, 0)),
            pl.BlockSpec((1, D), lambda i: (0, 0)),
        ],
        out_specs=pl.BlockSpec((G, D), lambda i: (0, 0)),
        out_shape=jax.ShapeDtypeStruct((G, D), jnp.float32),
        scratch_shapes=[pltpu.VMEM((G, D), jnp.float32)],
        interpret=_INTERPRET,
    )(src_r, edge_attr, smat, emat, lamT, gb)

    def _out_body(eb_ref, ea_ref, gamT_ref, padj_ref, out_ref):
        ebb = jnp.broadcast_to(eb_ref[0], (G, B))
        ind = ebb == lax.broadcasted_iota(jnp.int32, (G, B), 0)
        dense = jax.lax.dot_general(
            ea_ref[...].astype(jnp.bfloat16), gamT_ref[...],
            (((1,), (0,)), ((), ())),
            preferred_element_type=jnp.float32)
        unpool = jax.lax.dot_general(
            ind.astype(jnp.float32), padj_ref[...], (((0,), (0,)), ((), ())),
            preferred_element_type=jnp.float32)
        out_ref[...] = jnp.maximum(dense - unpool, 0.0)

    out = pl.pallas_call(
        _out_body,
        grid=(NB,),
        in_specs=[
            pl.BlockSpec((1, 1, B), lambda i: (i, 0, 0)),
            pl.BlockSpec((B, D), lambda i: (i, 0)),
            pl.BlockSpec((D, D), lambda i: (0, 0)),
            pl.BlockSpec((G, D), lambda i: (0, 0)),
        ],
        out_specs=pl.BlockSpec((B, D), lambda i: (i, 0)),
        out_shape=jax.ShapeDtypeStruct((E, D), jnp.float32),
        interpret=_INTERPRET,
    )(eb_r, edge_attr, gamT_bf, padj)
    return out


# speedup vs baseline: 1.0408x; 1.0408x over previous
import dataclasses
import functools

import jax
import jax.numpy as jnp
from jax import lax
from jax.experimental import pallas as pl
from jax.experimental.pallas import tpu as pltpu
from jax.experimental.pallas import tpu_sc as plsc

_G = 64
_B = 2560

_INTERPRET = False


def _edge_batch_sc(node_batch_i32, src_i32):
    N = node_batch_i32.shape[0]
    E = src_i32.shape[0]
    NC, NS, L = 2, 16, 16
    NW = NC * NS
    bpw = E // NW
    mesh = plsc.VectorSubcoreMesh(core_axis_name="c", subcore_axis_name="s")
    cp = pltpu.CompilerParams()
    if "needs_layout_passes" in pltpu.CompilerParams.__dataclass_fields__:
        cp = dataclasses.replace(cp, needs_layout_passes=False)

    @functools.partial(
        pl.kernel, mesh=mesh,
        compiler_params=cp,
        out_type=jax.ShapeDtypeStruct((E,), jnp.int32),
        scratch_types=[
            pltpu.VMEM((N,), jnp.int32),
            pltpu.VMEM((bpw,), jnp.int32),
            pltpu.VMEM((bpw,), jnp.int32),
            pltpu.SemaphoreType.DMA,
        ],
    )
    def k(nb_hbm, src_hbm, eb_hbm, nb_v, src_v, eb_v, sem):
        wid = lax.axis_index("s") * NC + lax.axis_index("c")
        base = wid * bpw
        pltpu.async_copy(nb_hbm, nb_v, sem).wait()
        pltpu.async_copy(src_hbm.at[pl.ds(base, bpw)], src_v, sem).wait()

        @pl.loop(0, bpw, step=L)
        def _(i):
            idx = src_v[pl.ds(i, L)]
            eb_v[pl.ds(i, L)] = plsc.load_gather(nb_v, [idx])

        pltpu.async_copy(eb_v, eb_hbm.at[pl.ds(base, bpw)], sem).wait()

    return k(node_batch_i32, src_i32)


def kernel(edge_attr, edge_index, node_batch, Gamma_W, Gamma_b, Lambda_W):
    E, D = edge_attr.shape
    G, B = _G, _B
    NB = E // B
    assert E % B == 0

    src = edge_index[0].astype(jnp.int32)
    nb32 = node_batch.astype(jnp.int32)
    eb = _edge_batch_sc(nb32, src)
    eb_r = eb.reshape(NB, 1, B)
    lamT = Lambda_W.T
    gamT_bf = Gamma_W.T.astype(jnp.bfloat16)
    gb = Gamma_b.reshape(1, D)

    def _pool_body(eb_ref, ea_ref, lamT_ref, gb_ref, padj_ref, acc_ref):
        i = pl.program_id(0)

        @pl.when(i == 0)
        def _():
            acc_ref[...] = jnp.zeros_like(acc_ref)

        ebb = jnp.broadcast_to(eb_ref[0], (G, B))
        ind = ebb == lax.broadcasted_iota(jnp.int32, (G, B), 0)
        indT = ind.astype(jnp.bfloat16)
        ea = ea_ref[...].astype(jnp.bfloat16)
        acc_ref[...] += jax.lax.dot_general(
            indT, ea, (((1,), (0,)), ((), ())),
            preferred_element_type=jnp.float32)

        @pl.when(i == NB - 1)
        def _():
            padj_ref[...] = jax.lax.dot_general(
                acc_ref[...], lamT_ref[...], (((1,), (0,)), ((), ())),
                preferred_element_type=jnp.float32) - gb_ref[...]

    padj = pl.pallas_call(
        _pool_body,
        grid=(NB,),
        in_specs=[
            pl.BlockSpec((1, 1, B), lambda i: (i, 0, 0)),
            pl.BlockSpec((B, D), lambda i: (i, 0)),
            pl.BlockSpec((D, D), lambda i: (0, 0)),
            pl.BlockSpec((1, D), lambda i: (0, 0)),
        ],
        out_specs=pl.BlockSpec((G, D), lambda i: (0, 0)),
        out_shape=jax.ShapeDtypeStruct((G, D), jnp.float32),
        scratch_shapes=[pltpu.VMEM((G, D), jnp.float32)],
        interpret=_INTERPRET,
    )(eb_r, edge_attr, lamT, gb)

    def _out_body(eb_ref, ea_ref, gamT_ref, padj_ref, out_ref):
        ebb = jnp.broadcast_to(eb_ref[0], (G, B))
        ind = ebb == lax.broadcasted_iota(jnp.int32, (G, B), 0)
        dense = jax.lax.dot_general(
            ea_ref[...].astype(jnp.bfloat16), gamT_ref[...],
            (((1,), (0,)), ((), ())),
            preferred_element_type=jnp.float32)
        unpool = jax.lax.dot_general(
            ind.astype(jnp.float32), padj_ref[...], (((0,), (0,)), ((), ())),
            preferred_element_type=jnp.float32)
        out_ref[...] = jnp.maximum(dense - unpool, 0.0)

    out = pl.pallas_call(
        _out_body,
        grid=(NB,),
        in_specs=[
            pl.BlockSpec((1, 1, B), lambda i: (i, 0, 0)),
            pl.BlockSpec((B, D), lambda i: (i, 0)),
            pl.BlockSpec((D, D), lambda i: (0, 0)),
            pl.BlockSpec((G, D), lambda i: (0, 0)),
        ],
        out_specs=pl.BlockSpec((B, D), lambda i: (i, 0)),
        out_shape=jax.ShapeDtypeStruct((E, D), jnp.float32),
        interpret=_INTERPRET,
    )(eb_r, edge_attr, gamT_bf, padj)
    return out


# speedup vs baseline: 1.4842x; 1.4260x over previous
import dataclasses
import functools

import jax
import jax.numpy as jnp
from jax import lax
from jax.experimental import pallas as pl
from jax.experimental.pallas import tpu as pltpu
from jax.experimental.pallas import tpu_sc as plsc

_G = 64
_B = 6400

_INTERPRET = False


def _edge_batch_sc(node_batch_i32, src_i32):
    N = node_batch_i32.shape[0]
    E = src_i32.shape[0]
    NC, NS, L = 2, 16, 16
    NW = NC * NS
    bpw = E // NW
    mesh = plsc.VectorSubcoreMesh(core_axis_name="c", subcore_axis_name="s")
    cp = pltpu.CompilerParams()
    if "needs_layout_passes" in pltpu.CompilerParams.__dataclass_fields__:
        cp = dataclasses.replace(cp, needs_layout_passes=False)

    @functools.partial(
        pl.kernel, mesh=mesh,
        compiler_params=cp,
        out_type=jax.ShapeDtypeStruct((E,), jnp.int32),
        scratch_types=[
            pltpu.VMEM((N,), jnp.int32),
            pltpu.VMEM((bpw,), jnp.int32),
            pltpu.VMEM((bpw,), jnp.int32),
            pltpu.SemaphoreType.DMA,
        ],
    )
    def k(nb_hbm, src_hbm, eb_hbm, nb_v, src_v, eb_v, sem):
        wid = lax.axis_index("s") * NC + lax.axis_index("c")
        base = wid * bpw
        pltpu.async_copy(nb_hbm, nb_v, sem).wait()
        pltpu.async_copy(src_hbm.at[pl.ds(base, bpw)], src_v, sem).wait()

        @pl.loop(0, bpw, step=L)
        def _(i):
            idx = src_v[pl.ds(i, L)]
            eb_v[pl.ds(i, L)] = plsc.load_gather(nb_v, [idx])

        pltpu.async_copy(eb_v, eb_hbm.at[pl.ds(base, bpw)], sem).wait()

    return k(node_batch_i32, src_i32)


def kernel(edge_attr, edge_index, node_batch, Gamma_W, Gamma_b, Lambda_W):
    E, D = edge_attr.shape
    G, B = _G, _B
    NB = E // B
    assert E % B == 0

    src = edge_index[0].astype(jnp.int32)
    nb32 = node_batch.astype(jnp.int32)
    eb = _edge_batch_sc(nb32, src)
    eb_r = eb.reshape(NB, 1, B)
    lamT = Lambda_W.T
    gamT_bf = Gamma_W.T.astype(jnp.bfloat16)
    gb = Gamma_b.reshape(1, D)

    def _pool_body(eb_ref, ea_ref, lamT_ref, gb_ref, padj_ref, acc_ref):
        i = pl.program_id(0)

        @pl.when(i == 0)
        def _():
            acc_ref[...] = jnp.zeros_like(acc_ref)

        ebb = jnp.broadcast_to(eb_ref[0], (G, B))
        ind = ebb == lax.broadcasted_iota(jnp.int32, (G, B), 0)
        indT = ind.astype(jnp.bfloat16)
        ea = ea_ref[...].astype(jnp.bfloat16)
        acc_ref[...] += jax.lax.dot_general(
            indT, ea, (((1,), (0,)), ((), ())),
            preferred_element_type=jnp.float32)

        @pl.when(i == NB - 1)
        def _():
            padj_ref[...] = jax.lax.dot_general(
                acc_ref[...], lamT_ref[...], (((1,), (0,)), ((), ())),
                preferred_element_type=jnp.float32) - gb_ref[...]

    padj = pl.pallas_call(
        _pool_body,
        grid=(NB,),
        in_specs=[
            pl.BlockSpec((1, 1, B), lambda i: (i, 0, 0)),
            pl.BlockSpec((B, D), lambda i: (i, 0)),
            pl.BlockSpec((D, D), lambda i: (0, 0)),
            pl.BlockSpec((1, D), lambda i: (0, 0)),
        ],
        out_specs=pl.BlockSpec((G, D), lambda i: (0, 0)),
        out_shape=jax.ShapeDtypeStruct((G, D), jnp.float32),
        scratch_shapes=[pltpu.VMEM((G, D), jnp.float32)],
        interpret=_INTERPRET,
    )(eb_r, edge_attr, lamT, gb)

    def _out_body(eb_ref, ea_ref, gamT_ref, padj_ref, out_ref):
        ebb = jnp.broadcast_to(eb_ref[0], (G, B))
        ind = ebb == lax.broadcasted_iota(jnp.int32, (G, B), 0)
        dense = jax.lax.dot_general(
            ea_ref[...].astype(jnp.bfloat16), gamT_ref[...],
            (((1,), (0,)), ((), ())),
            preferred_element_type=jnp.float32)
        unpool = jax.lax.dot_general(
            ind.astype(jnp.float32), padj_ref[...], (((0,), (0,)), ((), ())),
            preferred_element_type=jnp.float32)
        out_ref[...] = jnp.maximum(dense - unpool, 0.0)

    out = pl.pallas_call(
        _out_body,
        grid=(NB,),
        in_specs=[
            pl.BlockSpec((1, 1, B), lambda i: (i, 0, 0)),
            pl.BlockSpec((B, D), lambda i: (i, 0)),
            pl.BlockSpec((D, D), lambda i: (0, 0)),
            pl.BlockSpec((G, D), lambda i: (0, 0)),
        ],
        out_specs=pl.BlockSpec((B, D), lambda i: (i, 0)),
        out_shape=jax.ShapeDtypeStruct((E, D), jnp.float32),
        interpret=_INTERPRET,
    )(eb_r, edge_attr, gamT_bf, padj)
    return out


# speedup vs baseline: 1.6154x; 1.0884x over previous
import dataclasses
import functools

import jax
import jax.numpy as jnp
from jax import lax
from jax.experimental import pallas as pl
from jax.experimental.pallas import tpu as pltpu
from jax.experimental.pallas import tpu_sc as plsc

_G = 64
_B1 = 32000
_B2 = 16000

_INTERPRET = False


def _edge_batch_sc(node_batch_i32, src_i32):
    N = node_batch_i32.shape[0]
    E = src_i32.shape[0]
    NC, NS, L = 2, 16, 16
    NW = NC * NS
    bpw = E // NW
    mesh = plsc.VectorSubcoreMesh(core_axis_name="c", subcore_axis_name="s")
    cp = pltpu.CompilerParams()
    if "needs_layout_passes" in pltpu.CompilerParams.__dataclass_fields__:
        cp = dataclasses.replace(cp, needs_layout_passes=False)

    @functools.partial(
        pl.kernel, mesh=mesh,
        compiler_params=cp,
        out_type=jax.ShapeDtypeStruct((E,), jnp.int32),
        scratch_types=[
            pltpu.VMEM((N,), jnp.int32),
            pltpu.VMEM((bpw,), jnp.int32),
            pltpu.VMEM((bpw,), jnp.int32),
            pltpu.SemaphoreType.DMA,
        ],
    )
    def k(nb_hbm, src_hbm, eb_hbm, nb_v, src_v, eb_v, sem):
        wid = lax.axis_index("s") * NC + lax.axis_index("c")
        base = wid * bpw
        pltpu.async_copy(nb_hbm, nb_v, sem).wait()
        pltpu.async_copy(src_hbm.at[pl.ds(base, bpw)], src_v, sem).wait()

        @pl.loop(0, bpw, step=L)
        def _(i):
            idx = src_v[pl.ds(i, L)]
            eb_v[pl.ds(i, L)] = plsc.load_gather(nb_v, [idx])

        pltpu.async_copy(eb_v, eb_hbm.at[pl.ds(base, bpw)], sem).wait()

    return k(node_batch_i32, src_i32)


def kernel(edge_attr, edge_index, node_batch, Gamma_W, Gamma_b, Lambda_W):
    E, D = edge_attr.shape
    G, B1, B2 = _G, _B1, _B2
    NB1, NB2 = E // B1, E // B2
    assert E % B1 == 0 and E % B2 == 0

    src = edge_index[0].astype(jnp.int32)
    nb32 = node_batch.astype(jnp.int32)
    eb = _edge_batch_sc(nb32, src)
    eb_r1 = eb.reshape(NB1, 1, B1)
    eb_r2 = eb.reshape(NB2, 1, B2)
    lamT = Lambda_W.T
    gamT_bf = Gamma_W.T.astype(jnp.bfloat16)
    gb = Gamma_b.reshape(1, D)

    def _pool_body(eb_ref, ea_ref, lamT_ref, gb_ref, padj_ref, acc_ref):
        i = pl.program_id(0)

        @pl.when(i == 0)
        def _():
            acc_ref[...] = jnp.zeros_like(acc_ref)

        ebb = jnp.broadcast_to(eb_ref[0], (G, B1))
        ind = ebb == lax.broadcasted_iota(jnp.int32, (G, B1), 0)
        indT = ind.astype(jnp.bfloat16)
        ea = ea_ref[...].astype(jnp.bfloat16)
        acc_ref[...] += jax.lax.dot_general(
            indT, ea, (((1,), (0,)), ((), ())),
            preferred_element_type=jnp.float32)

        @pl.when(i == NB1 - 1)
        def _():
            padj_ref[...] = jax.lax.dot_general(
                acc_ref[...], lamT_ref[...], (((1,), (0,)), ((), ())),
                preferred_element_type=jnp.float32) - gb_ref[...]

    padj = pl.pallas_call(
        _pool_body,
        grid=(NB1,),
        in_specs=[
            pl.BlockSpec((1, 1, B1), lambda i: (i, 0, 0)),
            pl.BlockSpec((B1, D), lambda i: (i, 0)),
            pl.BlockSpec((D, D), lambda i: (0, 0)),
            pl.BlockSpec((1, D), lambda i: (0, 0)),
        ],
        out_specs=pl.BlockSpec((G, D), lambda i: (0, 0)),
        out_shape=jax.ShapeDtypeStruct((G, D), jnp.float32),
        scratch_shapes=[pltpu.VMEM((G, D), jnp.float32)],
        interpret=_INTERPRET,
    )(eb_r1, edge_attr, lamT, gb)

    def _out_body(eb_ref, ea_ref, gamT_ref, padj_ref, out_ref):
        ebb = jnp.broadcast_to(eb_ref[0], (G, B2))
        ind = ebb == lax.broadcasted_iota(jnp.int32, (G, B2), 0)
        dense = jax.lax.dot_general(
            ea_ref[...].astype(jnp.bfloat16), gamT_ref[...],
            (((1,), (0,)), ((), ())),
            preferred_element_type=jnp.float32)
        unpool = jax.lax.dot_general(
            ind.astype(jnp.float32), padj_ref[...], (((0,), (0,)), ((), ())),
            preferred_element_type=jnp.float32)
        out_ref[...] = jnp.maximum(dense - unpool, 0.0)

    out = pl.pallas_call(
        _out_body,
        grid=(NB2,),
        in_specs=[
            pl.BlockSpec((1, 1, B2), lambda i: (i, 0, 0)),
            pl.BlockSpec((B2, D), lambda i: (i, 0)),
            pl.BlockSpec((D, D), lambda i: (0, 0)),
            pl.BlockSpec((G, D), lambda i: (0, 0)),
        ],
        out_specs=pl.BlockSpec((B2, D), lambda i: (i, 0)),
        out_shape=jax.ShapeDtypeStruct((E, D), jnp.float32),
        interpret=_INTERPRET,
    )(eb_r2, edge_attr, gamT_bf, padj)
    return out


# speedup vs baseline: 1.6165x; 1.0007x over previous
import dataclasses
import functools

import jax
import jax.numpy as jnp
from jax import lax
from jax.experimental import pallas as pl
from jax.experimental.pallas import tpu as pltpu
from jax.experimental.pallas import tpu_sc as plsc

_G = 64
_B1 = 32000
_B2 = 16000

_INTERPRET = False


def _edge_batch_sc(node_batch_i32, src_i32):
    N = node_batch_i32.shape[0]
    E = src_i32.shape[0]
    NC, NS, L = 2, 16, 16
    NW = NC * NS
    bpw = E // NW
    mesh = plsc.VectorSubcoreMesh(core_axis_name="c", subcore_axis_name="s")
    cp = pltpu.CompilerParams()
    if "needs_layout_passes" in pltpu.CompilerParams.__dataclass_fields__:
        cp = dataclasses.replace(cp, needs_layout_passes=False)

    @functools.partial(
        pl.kernel, mesh=mesh,
        compiler_params=cp,
        out_type=jax.ShapeDtypeStruct((E,), jnp.int32),
        scratch_types=[
            pltpu.VMEM((N,), jnp.int32),
            pltpu.VMEM((bpw,), jnp.int32),
            pltpu.VMEM((bpw,), jnp.int32),
            pltpu.SemaphoreType.DMA,
        ],
    )
    def k(nb_hbm, src_hbm, eb_hbm, nb_v, src_v, eb_v, sem):
        wid = lax.axis_index("s") * NC + lax.axis_index("c")
        base = wid * bpw
        pltpu.async_copy(nb_hbm, nb_v, sem).wait()
        pltpu.async_copy(src_hbm.at[pl.ds(base, bpw)], src_v, sem).wait()

        @pl.loop(0, bpw, step=L)
        def _(i):
            idx = src_v[pl.ds(i, L)]
            eb_v[pl.ds(i, L)] = plsc.load_gather(nb_v, [idx])

        pltpu.async_copy(eb_v, eb_hbm.at[pl.ds(base, bpw)], sem).wait()

    return k(node_batch_i32, src_i32)


def kernel(edge_attr, edge_index, node_batch, Gamma_W, Gamma_b, Lambda_W):
    E, D = edge_attr.shape
    G, B1, B2 = _G, _B1, _B2
    NB1, NB2 = E // B1, E // B2
    assert E % B1 == 0 and E % B2 == 0

    eb = _edge_batch_sc(node_batch.astype(jnp.int32),
                        edge_index[0].astype(jnp.int32))
    eb_r1 = eb.reshape(NB1, 1, B1)
    eb_r2 = eb.reshape(NB2, 1, B2)
    gb = Gamma_b.reshape(1, D)

    def _pool_body(eb_ref, ea_ref, lamT_ref, gb_ref, padj_ref, acc_ref):
        i = pl.program_id(0)

        @pl.when(i == 0)
        def _():
            acc_ref[...] = jnp.zeros_like(acc_ref)

        ebb = jnp.broadcast_to(eb_ref[0], (G, B1))
        ind = ebb == lax.broadcasted_iota(jnp.int32, (G, B1), 0)
        indT = ind.astype(jnp.bfloat16)
        ea = ea_ref[...].astype(jnp.bfloat16)
        acc_ref[...] += jax.lax.dot_general(
            indT, ea, (((1,), (0,)), ((), ())),
            preferred_element_type=jnp.float32)

        @pl.when(i == NB1 - 1)
        def _():
            padj_ref[...] = jax.lax.dot_general(
                acc_ref[...], lamT_ref[...], (((1,), (1,)), ((), ())),
                preferred_element_type=jnp.float32) - gb_ref[...]

    padj = pl.pallas_call(
        _pool_body,
        grid=(NB1,),
        in_specs=[
            pl.BlockSpec((1, 1, B1), lambda i: (i, 0, 0)),
            pl.BlockSpec((B1, D), lambda i: (i, 0)),
            pl.BlockSpec((D, D), lambda i: (0, 0)),
            pl.BlockSpec((1, D), lambda i: (0, 0)),
        ],
        out_specs=pl.BlockSpec((G, D), lambda i: (0, 0)),
        out_shape=jax.ShapeDtypeStruct((G, D), jnp.float32),
        scratch_shapes=[pltpu.VMEM((G, D), jnp.float32)],
        interpret=_INTERPRET,
    )(eb_r1, edge_attr, Lambda_W, gb)

    def _out_body(eb_ref, ea_ref, gam_ref, padj_ref, out_ref):
        ebb = jnp.broadcast_to(eb_ref[0], (G, B2))
        ind = ebb == lax.broadcasted_iota(jnp.int32, (G, B2), 0)
        dense = jax.lax.dot_general(
            ea_ref[...].astype(jnp.bfloat16), gam_ref[...].astype(jnp.bfloat16),
            (((1,), (1,)), ((), ())),
            preferred_element_type=jnp.float32)
        unpool = jax.lax.dot_general(
            ind.astype(jnp.float32), padj_ref[...], (((0,), (0,)), ((), ())),
            preferred_element_type=jnp.float32)
        out_ref[...] = jnp.maximum(dense - unpool, 0.0)

    out = pl.pallas_call(
        _out_body,
        grid=(NB2,),
        in_specs=[
            pl.BlockSpec((1, 1, B2), lambda i: (i, 0, 0)),
            pl.BlockSpec((B2, D), lambda i: (i, 0)),
            pl.BlockSpec((D, D), lambda i: (0, 0)),
            pl.BlockSpec((G, D), lambda i: (0, 0)),
        ],
        out_specs=pl.BlockSpec((B2, D), lambda i: (i, 0)),
        out_shape=jax.ShapeDtypeStruct((E, D), jnp.float32),
        interpret=_INTERPRET,
    )(eb_r2, edge_attr, Gamma_W, padj)
    return out
